# Initial kernel scaffold; baseline (speedup 1.0000x reference)
import jax
import jax.numpy as jnp
from jax.experimental import pallas as pl


def kernel(pixels_cls_scores, targets):
    raise NotImplementedError("write your pallas kernel here")


# speedup vs baseline: 3.4729x; 3.4729x over previous
import jax
import jax.numpy as jnp
from jax import lax
from jax.experimental import pallas as pl
from jax.experimental.pallas import tpu as pltpu

_N, _K, _H, _W = 128, 9, 64, 32
_P = _H * _W
_TOTAL = _N * _P
_KEEP = _TOTAL // 2
_LS = 0.1


def _body(scores_ref, tgt_ref, loss_out, acc_out, bits_ref, acc_ref):
    i = pl.program_id(0)
    s = scores_ref[0]
    t = tgt_ref[0]

    m = jnp.max(s, axis=0, keepdims=True)
    se = jnp.sum(jnp.exp(s - m), axis=0, keepdims=True)
    lse = jnp.log(se) + m
    kio = lax.broadcasted_iota(jnp.int32, (_K, _P), 0)
    onehot = kio == t
    s_t = jnp.sum(jnp.where(onehot, s, 0.0), axis=0, keepdims=True)
    mean_s = jnp.mean(s, axis=0, keepdims=True)
    loss = lse - (1.0 - _LS) * s_t - _LS * mean_s
    bits_ref[i] = lax.bitcast_convert_type(loss, jnp.int32)

    idx = jnp.min(jnp.where(s == m, kio, _K), axis=0, keepdims=True)
    correct = jnp.sum((idx == t).astype(jnp.float32))

    @pl.when(i == 0)
    def _():
        acc_ref[0, 0] = 0.0

    acc_ref[0, 0] += correct

    @pl.when(i == _N - 1)
    def _():
        bits = bits_ref[:, 0, :]
        keep = jnp.int32(_KEEP)

        def step(j, prefix):
            cand = prefix | lax.shift_left(jnp.int32(1), 30 - j)
            cnt = jnp.sum((bits < cand).astype(jnp.int32))
            return jnp.where(cnt < keep, cand, prefix)

        tbits = lax.fori_loop(0, 31, step, jnp.int32(0))
        tval = lax.bitcast_convert_type(tbits, jnp.float32)
        lt = bits < tbits
        vals = lax.bitcast_convert_type(bits, jnp.float32)
        sum_lt = jnp.sum(jnp.where(lt, vals, 0.0))
        cnt_lt = jnp.sum(lt.astype(jnp.int32))
        loss_out[0, 0] = (sum_lt + tval * (keep - cnt_lt).astype(jnp.float32)) / _KEEP
        acc_out[0, 0] = acc_ref[0, 0] / _TOTAL


def kernel(pixels_cls_scores, targets):
    scores = pixels_cls_scores.reshape(_N, _K, _P)
    tgt = targets.reshape(_N, 1, _P)
    loss, acc = pl.pallas_call(
        _body,
        grid=(_N,),
        in_specs=[
            pl.BlockSpec((1, _K, _P), lambda i: (i, 0, 0)),
            pl.BlockSpec((1, 1, _P), lambda i: (i, 0, 0)),
        ],
        out_specs=[
            pl.BlockSpec((1, 1), lambda i: (0, 0), memory_space=pltpu.SMEM),
            pl.BlockSpec((1, 1), lambda i: (0, 0), memory_space=pltpu.SMEM),
        ],
        out_shape=[
            jax.ShapeDtypeStruct((1, 1), jnp.float32),
            jax.ShapeDtypeStruct((1, 1), jnp.float32),
        ],
        scratch_shapes=[
            pltpu.VMEM((_N, 1, _P), jnp.int32),
            pltpu.SMEM((1, 1), jnp.float32),
        ],
    )(scores, tgt)
    return loss[0, 0], acc[0, 0]


# speedup vs baseline: 4.5280x; 1.3038x over previous
import jax
import jax.numpy as jnp
from jax import lax
from jax.experimental import pallas as pl
from jax.experimental.pallas import tpu as pltpu

_N, _K, _H, _W = 128, 9, 64, 32
_P = _H * _W
_R = 4
_G = _N // _R
_SL, _LN = 8, _P // 8
_TOTAL = _N * _P
_KEEP = _TOTAL // 2
_LS = 0.1


def _body(scores_ref, tgt_ref, loss_out, acc_out,
          bits_ref, acc_ref, minb_ref, maxb_ref):
    i = pl.program_id(0)
    s = scores_ref[0].reshape(_R, _K, _SL, _LN)
    t = tgt_ref[0]

    m = jnp.max(s, axis=1, keepdims=True)
    se = jnp.sum(jnp.exp(s - m), axis=1)
    lse = jnp.log(se) + m[:, 0]
    kio = lax.broadcasted_iota(jnp.int32, (_R, _K, _SL, _LN), 1)
    onehot = kio == t[:, None]
    s_t = jnp.sum(jnp.where(onehot, s, 0.0), axis=1)
    mean_s = jnp.mean(s, axis=1)
    loss = lse - (1.0 - _LS) * s_t - _LS * mean_s
    bits = lax.bitcast_convert_type(loss, jnp.int32)
    bits_ref[i] = bits

    idx = jnp.min(jnp.where(s == m, kio, _K), axis=1)
    correct = (idx == t).astype(jnp.float32)

    @pl.when(i == 0)
    def _():
        acc_ref[...] = jnp.zeros_like(acc_ref)
        minb_ref[...] = jnp.full_like(minb_ref, jnp.int32(0x7FFFFFFF))
        maxb_ref[...] = jnp.zeros_like(maxb_ref)

    acc_ref[...] += correct
    minb_ref[...] = jnp.minimum(minb_ref[...], bits)
    maxb_ref[...] = jnp.maximum(maxb_ref[...], bits)

    @pl.when(i == _G - 1)
    def _():
        allb = bits_ref[...]
        minb = jnp.min(minb_ref[...])
        maxb = jnp.max(maxb_ref[...])
        keep = jnp.int32(_KEEP)

        def full_count(cand):
            return jnp.sum((allb < cand).astype(jnp.int32))

        def step(j, prefix):
            cand = prefix | lax.shift_left(jnp.int32(1), 30 - j)
            inside = (cand > minb) & (cand <= maxb)
            cnt = lax.cond(
                inside,
                lambda: full_count(cand),
                lambda: jnp.where(cand <= minb, jnp.int32(0),
                                  jnp.int32(_TOTAL)),
            )
            return jnp.where(cnt < keep, cand, prefix)

        tbits = lax.fori_loop(0, 31, step, jnp.int32(0))
        tval = lax.bitcast_convert_type(tbits, jnp.float32)
        lt = allb < tbits
        vals = lax.bitcast_convert_type(allb, jnp.float32)
        sum_lt = jnp.sum(jnp.where(lt, vals, 0.0))
        cnt_lt = jnp.sum(lt.astype(jnp.int32))
        loss_out[0, 0] = (sum_lt + tval * (keep - cnt_lt).astype(jnp.float32)) / _KEEP
        acc_out[0, 0] = jnp.sum(acc_ref[...]) / _TOTAL


def kernel(pixels_cls_scores, targets):
    scores = pixels_cls_scores.reshape(_G, _R * _K, _SL, _LN)
    tgt = targets.reshape(_G, _R, _SL, _LN)
    loss, acc = pl.pallas_call(
        _body,
        grid=(_G,),
        in_specs=[
            pl.BlockSpec((1, _R * _K, _SL, _LN), lambda i: (i, 0, 0, 0)),
            pl.BlockSpec((1, _R, _SL, _LN), lambda i: (i, 0, 0, 0)),
        ],
        out_specs=[
            pl.BlockSpec((1, 1), lambda i: (0, 0), memory_space=pltpu.SMEM),
            pl.BlockSpec((1, 1), lambda i: (0, 0), memory_space=pltpu.SMEM),
        ],
        out_shape=[
            jax.ShapeDtypeStruct((1, 1), jnp.float32),
            jax.ShapeDtypeStruct((1, 1), jnp.float32),
        ],
        scratch_shapes=[
            pltpu.VMEM((_G, _R, _SL, _LN), jnp.int32),
            pltpu.VMEM((_R, _SL, _LN), jnp.float32),
            pltpu.VMEM((_R, _SL, _LN), jnp.int32),
            pltpu.VMEM((_R, _SL, _LN), jnp.int32),
        ],
    )(scores, tgt)
    return loss[0, 0], acc[0, 0]


# speedup vs baseline: 5.0524x; 1.1158x over previous
import jax
import jax.numpy as jnp
from jax import lax
from jax.experimental import pallas as pl
from jax.experimental.pallas import tpu as pltpu

_N, _K, _H, _W = 128, 9, 64, 32
_P = _H * _W
_R = 4
_G = _N // _R
_SL, _LN = 8, _P // 8
_TOTAL = _N * _P
_KEEP = _TOTAL // 2
_LS = 0.1


def _body(scores_ref, tgt_ref, loss_out, acc_out,
          bits_ref, acc_ref, minb_ref, maxb_ref):
    i = pl.program_id(0)
    s = scores_ref[0].reshape(_R, _K, _H, _W)
    t = tgt_ref[0]

    m = jnp.max(s, axis=1, keepdims=True)
    se = jnp.sum(jnp.exp(s - m), axis=1)
    lse = jnp.log(se) + m[:, 0]
    kio = lax.broadcasted_iota(jnp.int32, (_R, _K, _H, _W), 1)
    onehot = kio == t[:, None]
    s_t = jnp.sum(jnp.where(onehot, s, 0.0), axis=1)
    mean_s = jnp.mean(s, axis=1)
    loss = lse - (1.0 - _LS) * s_t - _LS * mean_s
    compact = jnp.concatenate(
        [loss[:, 8 * j:8 * (j + 1), :] for j in range(8)], axis=2)
    bits = lax.bitcast_convert_type(compact, jnp.int32)
    bits_ref[i] = bits

    idx = jnp.min(jnp.where(s == m, kio, _K), axis=1)
    correct = (idx == t).astype(jnp.float32)

    @pl.when(i == 0)
    def _():
        acc_ref[...] = jnp.zeros_like(acc_ref)
        minb_ref[...] = jnp.full_like(minb_ref, jnp.int32(0x7FFFFFFF))
        maxb_ref[...] = jnp.zeros_like(maxb_ref)

    acc_ref[...] += correct
    minb_ref[...] = jnp.minimum(minb_ref[...], bits)
    maxb_ref[...] = jnp.maximum(maxb_ref[...], bits)

    @pl.when(i == _G - 1)
    def _():
        allb = bits_ref[...]
        minb = jnp.min(minb_ref[...])
        maxb = jnp.max(maxb_ref[...])
        keep = jnp.int32(_KEEP)

        def full_count(cand):
            return jnp.sum((allb < cand).astype(jnp.int32))

        def step(j, prefix):
            cand = prefix | lax.shift_left(jnp.int32(1), 30 - j)
            inside = (cand > minb) & (cand <= maxb)
            cnt = lax.cond(
                inside,
                lambda: full_count(cand),
                lambda: jnp.where(cand <= minb, jnp.int32(0),
                                  jnp.int32(_TOTAL)),
            )
            return jnp.where(cnt < keep, cand, prefix)

        tbits = lax.fori_loop(0, 31, step, jnp.int32(0))
        tval = lax.bitcast_convert_type(tbits, jnp.float32)
        lt = allb < tbits
        vals = lax.bitcast_convert_type(allb, jnp.float32)
        sum_lt = jnp.sum(jnp.where(lt, vals, 0.0))
        cnt_lt = jnp.sum(lt.astype(jnp.int32))
        loss_out[0, 0] = (sum_lt + tval * (keep - cnt_lt).astype(jnp.float32)) / _KEEP
        acc_out[0, 0] = jnp.sum(acc_ref[...]) / _TOTAL


def kernel(pixels_cls_scores, targets):
    scores = pixels_cls_scores.reshape(_G, _R * _K, _H, _W)
    tgt = targets.reshape(_G, _R, _H, _W)
    loss, acc = pl.pallas_call(
        _body,
        grid=(_G,),
        in_specs=[
            pl.BlockSpec((1, _R * _K, _H, _W), lambda i: (i, 0, 0, 0)),
            pl.BlockSpec((1, _R, _H, _W), lambda i: (i, 0, 0, 0)),
        ],
        out_specs=[
            pl.BlockSpec((1, 1), lambda i: (0, 0), memory_space=pltpu.SMEM),
            pl.BlockSpec((1, 1), lambda i: (0, 0), memory_space=pltpu.SMEM),
        ],
        out_shape=[
            jax.ShapeDtypeStruct((1, 1), jnp.float32),
            jax.ShapeDtypeStruct((1, 1), jnp.float32),
        ],
        scratch_shapes=[
            pltpu.VMEM((_G, _R, _SL, _LN), jnp.int32),
            pltpu.VMEM((_R, _H, _W), jnp.float32),
            pltpu.VMEM((_R, _SL, _LN), jnp.int32),
            pltpu.VMEM((_R, _SL, _LN), jnp.int32),
        ],
    )(scores, tgt)
    return loss[0, 0], acc[0, 0]


# speedup vs baseline: 19.6914x; 3.8975x over previous
import jax
import jax.numpy as jnp
from jax import lax
from jax.experimental import pallas as pl
from jax.experimental.pallas import tpu as pltpu

_N, _K, _H, _W = 128, 9, 64, 32
_HB = 8
_G = _H // _HB
_TOTAL = _N * _H * _W
_KEEP = _TOTAL // 2
_LS = 0.1


def _body(scores_ref, tgt_ref, loss_out, acc_out,
          bits_ref, acc_ref, minb_ref, maxb_ref):
    i = pl.program_id(0)
    s = scores_ref[...]
    t = tgt_ref[...]

    m = jnp.max(s, axis=0)
    se = jnp.sum(jnp.exp(s - m[None]), axis=0)
    lse = jnp.log(se) + m
    kio = lax.broadcasted_iota(jnp.int32, (_K, _HB, _W, _N), 0)
    onehot = kio == t[None]
    s_t = jnp.sum(jnp.where(onehot, s, 0.0), axis=0)
    mean_s = jnp.mean(s, axis=0)
    loss = lse - (1.0 - _LS) * s_t - _LS * mean_s
    bits = lax.bitcast_convert_type(loss, jnp.int32)
    bits_ref[i] = bits

    idx = jnp.min(jnp.where(s == m[None], kio, _K), axis=0)
    correct = (idx == t).astype(jnp.float32)

    @pl.when(i == 0)
    def _():
        acc_ref[...] = jnp.zeros_like(acc_ref)
        minb_ref[...] = jnp.full_like(minb_ref, jnp.int32(0x7FFFFFFF))
        maxb_ref[...] = jnp.zeros_like(maxb_ref)

    acc_ref[...] += correct
    minb_ref[...] = jnp.minimum(minb_ref[...], bits)
    maxb_ref[...] = jnp.maximum(maxb_ref[...], bits)

    @pl.when(i == _G - 1)
    def _():
        allb = bits_ref[...]
        minb = jnp.min(minb_ref[...])
        maxb = jnp.max(maxb_ref[...])
        keep = jnp.int32(_KEEP)

        def full_count(cand):
            return jnp.sum((allb < cand).astype(jnp.int32))

        def step(j, prefix):
            cand = prefix | lax.shift_left(jnp.int32(1), 30 - j)
            inside = (cand > minb) & (cand <= maxb)
            cnt = lax.cond(
                inside,
                lambda: full_count(cand),
                lambda: jnp.where(cand <= minb, jnp.int32(0),
                                  jnp.int32(_TOTAL)),
            )
            return jnp.where(cnt < keep, cand, prefix)

        tbits = lax.fori_loop(0, 31, step, jnp.int32(0))
        tval = lax.bitcast_convert_type(tbits, jnp.float32)
        lt = allb < tbits
        vals = lax.bitcast_convert_type(allb, jnp.float32)
        sum_lt = jnp.sum(jnp.where(lt, vals, 0.0))
        cnt_lt = jnp.sum(lt.astype(jnp.int32))
        loss_out[0, 0] = (sum_lt + tval * (keep - cnt_lt).astype(jnp.float32)) / _KEEP
        acc_out[0, 0] = jnp.sum(acc_ref[...]) / _TOTAL


def kernel(pixels_cls_scores, targets):
    scores = jnp.transpose(pixels_cls_scores, (1, 2, 3, 0))
    tgt = jnp.transpose(targets, (1, 2, 0))
    loss, acc = pl.pallas_call(
        _body,
        grid=(_G,),
        in_specs=[
            pl.BlockSpec((_K, _HB, _W, _N), lambda i: (0, i, 0, 0)),
            pl.BlockSpec((_HB, _W, _N), lambda i: (i, 0, 0)),
        ],
        out_specs=[
            pl.BlockSpec((1, 1), lambda i: (0, 0), memory_space=pltpu.SMEM),
            pl.BlockSpec((1, 1), lambda i: (0, 0), memory_space=pltpu.SMEM),
        ],
        out_shape=[
            jax.ShapeDtypeStruct((1, 1), jnp.float32),
            jax.ShapeDtypeStruct((1, 1), jnp.float32),
        ],
        scratch_shapes=[
            pltpu.VMEM((_G, _HB, _W, _N), jnp.int32),
            pltpu.VMEM((_HB, _W, _N), jnp.float32),
            pltpu.VMEM((_HB, _W, _N), jnp.int32),
            pltpu.VMEM((_HB, _W, _N), jnp.int32),
        ],
    )(scores, tgt)
    return loss[0, 0], acc[0, 0]


# speedup vs baseline: 23.0041x; 1.1682x over previous
import jax
import jax.numpy as jnp
from jax import lax
from jax.experimental import pallas as pl
from jax.experimental.pallas import tpu as pltpu

_N, _K, _H, _W = 128, 9, 64, 32
_HB = 8
_G = _H // _HB
_TOTAL = _N * _H * _W
_KEEP = _TOTAL // 2
_LS = 0.1


def _body(scores_ref, tgt_ref, loss_out, acc_out,
          bits_ref, acc_ref, minb_ref, maxb_ref):
    i = pl.program_id(0)
    s = scores_ref[...]
    t = tgt_ref[...]

    m = jnp.max(s, axis=0)
    se = jnp.sum(jnp.exp(s - m[None]), axis=0)
    lse = jnp.log(se) + m
    kio = lax.broadcasted_iota(jnp.int32, (_K, _HB, _W, _N), 0)
    onehot = kio == t[None]
    s_t = jnp.sum(jnp.where(onehot, s, 0.0), axis=0)
    mean_s = jnp.mean(s, axis=0)
    loss = lse - (1.0 - _LS) * s_t - _LS * mean_s
    bits = lax.bitcast_convert_type(loss, jnp.int32)
    bits_ref[i] = bits

    idx = jnp.min(jnp.where(s == m[None], kio, _K), axis=0)
    correct = (idx == t).astype(jnp.float32)

    @pl.when(i == 0)
    def _():
        acc_ref[...] = jnp.zeros_like(acc_ref)
        minb_ref[...] = jnp.full_like(minb_ref, jnp.int32(0x7FFFFFFF))
        maxb_ref[...] = jnp.zeros_like(maxb_ref)

    acc_ref[...] += correct
    minb_ref[...] = jnp.minimum(minb_ref[...], bits)
    maxb_ref[...] = jnp.maximum(maxb_ref[...], bits)

    @pl.when(i == _G - 1)
    def _():
        allb = bits_ref[...]
        minb = jnp.min(minb_ref[...])
        maxb = jnp.max(maxb_ref[...])
        keep = jnp.int32(_KEEP)

        def tree_total(x):
            parts = [x[g] for g in range(_G)]
            while len(parts) > 1:
                parts = [a + b for a, b in zip(parts[::2], parts[1::2])]
            return jnp.sum(parts[0])

        def full_count(cand):
            return tree_total((allb < cand).astype(jnp.int32))

        def step(j, prefix):
            cand = prefix | lax.shift_left(jnp.int32(1), 30 - j)
            inside = (cand > minb) & (cand <= maxb)
            cnt = lax.cond(
                inside,
                lambda: full_count(cand),
                lambda: jnp.where(cand <= minb, jnp.int32(0),
                                  jnp.int32(_TOTAL)),
            )
            return jnp.where(cnt < keep, cand, prefix)

        tbits = lax.fori_loop(0, 31, step, jnp.int32(0))
        tval = lax.bitcast_convert_type(tbits, jnp.float32)
        lt = allb < tbits
        vals = lax.bitcast_convert_type(allb, jnp.float32)
        sum_lt = tree_total(jnp.where(lt, vals, 0.0))
        cnt_lt = tree_total(lt.astype(jnp.int32))
        loss_out[0, 0] = (sum_lt + tval * (keep - cnt_lt).astype(jnp.float32)) / _KEEP
        acc_out[0, 0] = jnp.sum(acc_ref[...]) / _TOTAL


def kernel(pixels_cls_scores, targets):
    scores = jnp.transpose(pixels_cls_scores, (1, 2, 3, 0))
    tgt = jnp.transpose(targets, (1, 2, 0))
    loss, acc = pl.pallas_call(
        _body,
        grid=(_G,),
        in_specs=[
            pl.BlockSpec((_K, _HB, _W, _N), lambda i: (0, i, 0, 0)),
            pl.BlockSpec((_HB, _W, _N), lambda i: (i, 0, 0)),
        ],
        out_specs=[
            pl.BlockSpec((1, 1), lambda i: (0, 0), memory_space=pltpu.SMEM),
            pl.BlockSpec((1, 1), lambda i: (0, 0), memory_space=pltpu.SMEM),
        ],
        out_shape=[
            jax.ShapeDtypeStruct((1, 1), jnp.float32),
            jax.ShapeDtypeStruct((1, 1), jnp.float32),
        ],
        scratch_shapes=[
            pltpu.VMEM((_G, _HB, _W, _N), jnp.int32),
            pltpu.VMEM((_HB, _W, _N), jnp.float32),
            pltpu.VMEM((_HB, _W, _N), jnp.int32),
            pltpu.VMEM((_HB, _W, _N), jnp.int32),
        ],
    )(scores, tgt)
    return loss[0, 0], acc[0, 0]


# speedup vs baseline: 24.7230x; 1.0747x over previous
import jax
import jax.numpy as jnp
from jax import lax
from jax.experimental import pallas as pl
from jax.experimental.pallas import tpu as pltpu

_N, _K, _H, _W = 128, 9, 64, 32
_HB = 8
_G = _H // _HB
_TOTAL = _N * _H * _W
_KEEP = _TOTAL // 2
_LS = 0.1


def _body(scores_ref, tgt_ref, loss_out, acc_out,
          bits_ref, acc_ref, minb_ref, maxb_ref):
    i = pl.program_id(0)
    s = scores_ref[...]
    t = tgt_ref[...]

    m = jnp.max(s, axis=0)
    se = jnp.sum(jnp.exp(s - m[None]), axis=0)
    lse = jnp.log(se) + m
    kio = lax.broadcasted_iota(jnp.int32, (_K, _HB, _W, _N), 0)
    onehot = kio == t[None]
    s_t = jnp.sum(jnp.where(onehot, s, 0.0), axis=0)
    mean_s = jnp.mean(s, axis=0)
    loss = lse - (1.0 - _LS) * s_t - _LS * mean_s
    bits = lax.bitcast_convert_type(loss, jnp.int32)
    bits_ref[i] = bits

    idx = jnp.min(jnp.where(s == m[None], kio, _K), axis=0)
    correct = (idx == t).astype(jnp.float32)

    @pl.when(i == 0)
    def _():
        acc_ref[...] = jnp.zeros_like(acc_ref)
        minb_ref[...] = jnp.full_like(minb_ref, jnp.int32(0x7FFFFFFF))
        maxb_ref[...] = jnp.zeros_like(maxb_ref)

    acc_ref[...] += correct
    minb_ref[...] = jnp.minimum(minb_ref[...], bits)
    maxb_ref[...] = jnp.maximum(maxb_ref[...], bits)

    @pl.when(i == _G - 1)
    def _():
        nv = _G * _HB * _W // 8
        allb = bits_ref[...].reshape(nv, 8, _N)
        minb = jnp.min(minb_ref[...])
        maxb = jnp.max(maxb_ref[...])
        keep = jnp.int32(_KEEP)

        def full_count(cand):
            accs = [jnp.zeros((8, _N), jnp.int32) for _ in range(4)]
            for g in range(nv):
                accs[g % 4] = accs[g % 4] + (allb[g] < cand).astype(jnp.int32)
            return jnp.sum((accs[0] + accs[1]) + (accs[2] + accs[3]))

        def step(j, prefix):
            cand = prefix | lax.shift_left(jnp.int32(1), 30 - j)
            inside = (cand > minb) & (cand <= maxb)
            cnt = lax.cond(
                inside,
                lambda: full_count(cand),
                lambda: jnp.where(cand <= minb, jnp.int32(0),
                                  jnp.int32(_TOTAL)),
            )
            return jnp.where(cnt < keep, cand, prefix)

        tbits = lax.fori_loop(0, 31, step, jnp.int32(0))
        tval = lax.bitcast_convert_type(tbits, jnp.float32)
        cacc = [jnp.zeros((8, _N), jnp.int32) for _ in range(4)]
        sacc = [jnp.zeros((8, _N), jnp.float32) for _ in range(4)]
        for g in range(nv):
            m = allb[g] < tbits
            v = lax.bitcast_convert_type(allb[g], jnp.float32)
            cacc[g % 4] = cacc[g % 4] + m.astype(jnp.int32)
            sacc[g % 4] = sacc[g % 4] + jnp.where(m, v, 0.0)
        cnt_lt = jnp.sum((cacc[0] + cacc[1]) + (cacc[2] + cacc[3]))
        sum_lt = jnp.sum((sacc[0] + sacc[1]) + (sacc[2] + sacc[3]))
        loss_out[0, 0] = (sum_lt + tval * (keep - cnt_lt).astype(jnp.float32)) / _KEEP
        acc_out[0, 0] = jnp.sum(acc_ref[...]) / _TOTAL


def kernel(pixels_cls_scores, targets):
    scores = jnp.transpose(pixels_cls_scores, (1, 2, 3, 0))
    tgt = jnp.transpose(targets, (1, 2, 0))
    loss, acc = pl.pallas_call(
        _body,
        grid=(_G,),
        in_specs=[
            pl.BlockSpec((_K, _HB, _W, _N), lambda i: (0, i, 0, 0)),
            pl.BlockSpec((_HB, _W, _N), lambda i: (i, 0, 0)),
        ],
        out_specs=[
            pl.BlockSpec((1, 1), lambda i: (0, 0), memory_space=pltpu.SMEM),
            pl.BlockSpec((1, 1), lambda i: (0, 0), memory_space=pltpu.SMEM),
        ],
        out_shape=[
            jax.ShapeDtypeStruct((1, 1), jnp.float32),
            jax.ShapeDtypeStruct((1, 1), jnp.float32),
        ],
        scratch_shapes=[
            pltpu.VMEM((_G, _HB, _W, _N), jnp.int32),
            pltpu.VMEM((_HB, _W, _N), jnp.float32),
            pltpu.VMEM((_HB, _W, _N), jnp.int32),
            pltpu.VMEM((_HB, _W, _N), jnp.int32),
        ],
    )(scores, tgt)
    return loss[0, 0], acc[0, 0]


# speedup vs baseline: 27.0192x; 1.0929x over previous
import jax
import jax.numpy as jnp
from jax import lax
from jax.experimental import pallas as pl
from jax.experimental.pallas import tpu as pltpu

_N, _K, _H, _W = 128, 9, 64, 32
_HB = 16
_G = _H // _HB
_TOTAL = _N * _H * _W
_KEEP = _TOTAL // 2
_LS = 0.1


def _body(scores_ref, tgt_ref, loss_out, acc_out,
          bits_ref, acc_ref, minb_ref, maxb_ref):
    i = pl.program_id(0)
    s = scores_ref[...]
    t = tgt_ref[...]

    m = jnp.max(s, axis=0)
    se = jnp.sum(jnp.exp(s - m[None]), axis=0)
    lse = jnp.log(se) + m
    kio = lax.broadcasted_iota(jnp.int32, (_K, _HB, _W, _N), 0)
    onehot = kio == t[None]
    s_t = jnp.sum(jnp.where(onehot, s, 0.0), axis=0)
    mean_s = jnp.mean(s, axis=0)
    loss = lse - (1.0 - _LS) * s_t - _LS * mean_s
    bits = lax.bitcast_convert_type(loss, jnp.int32)
    bits_ref[i] = bits

    idx = jnp.min(jnp.where(s == m[None], kio, _K), axis=0)
    correct = (idx == t).astype(jnp.float32)

    @pl.when(i == 0)
    def _():
        acc_ref[...] = jnp.zeros_like(acc_ref)
        minb_ref[...] = jnp.full_like(minb_ref, jnp.int32(0x7FFFFFFF))
        maxb_ref[...] = jnp.zeros_like(maxb_ref)

    acc_ref[...] += correct
    minb_ref[...] = jnp.minimum(minb_ref[...], bits)
    maxb_ref[...] = jnp.maximum(maxb_ref[...], bits)

    @pl.when(i == _G - 1)
    def _():
        nv = _G * _HB * _W // 8
        allb = bits_ref[...].reshape(nv, 8, _N)
        minb = jnp.min(minb_ref[...])
        maxb = jnp.max(maxb_ref[...])
        keep = jnp.int32(_KEEP)

        def full_count(cand):
            accs = [jnp.zeros((8, _N), jnp.int32) for _ in range(4)]
            for g in range(nv):
                accs[g % 4] = accs[g % 4] + (allb[g] < cand).astype(jnp.int32)
            return jnp.sum((accs[0] + accs[1]) + (accs[2] + accs[3]))

        def step(j, prefix):
            cand = prefix | lax.shift_left(jnp.int32(1), 30 - j)
            inside = (cand > minb) & (cand <= maxb)
            cnt = lax.cond(
                inside,
                lambda: full_count(cand),
                lambda: jnp.where(cand <= minb, jnp.int32(0),
                                  jnp.int32(_TOTAL)),
            )
            return jnp.where(cnt < keep, cand, prefix)

        tbits = lax.fori_loop(0, 31, step, jnp.int32(0))
        tval = lax.bitcast_convert_type(tbits, jnp.float32)
        cacc = [jnp.zeros((8, _N), jnp.int32) for _ in range(4)]
        sacc = [jnp.zeros((8, _N), jnp.float32) for _ in range(4)]
        for g in range(nv):
            m = allb[g] < tbits
            v = lax.bitcast_convert_type(allb[g], jnp.float32)
            cacc[g % 4] = cacc[g % 4] + m.astype(jnp.int32)
            sacc[g % 4] = sacc[g % 4] + jnp.where(m, v, 0.0)
        cnt_lt = jnp.sum((cacc[0] + cacc[1]) + (cacc[2] + cacc[3]))
        sum_lt = jnp.sum((sacc[0] + sacc[1]) + (sacc[2] + sacc[3]))
        loss_out[0, 0] = (sum_lt + tval * (keep - cnt_lt).astype(jnp.float32)) / _KEEP
        acc_out[0, 0] = jnp.sum(acc_ref[...]) / _TOTAL


def kernel(pixels_cls_scores, targets):
    scores = jnp.transpose(pixels_cls_scores, (1, 2, 3, 0))
    tgt = jnp.transpose(targets, (1, 2, 0))
    loss, acc = pl.pallas_call(
        _body,
        grid=(_G,),
        in_specs=[
            pl.BlockSpec((_K, _HB, _W, _N), lambda i: (0, i, 0, 0)),
            pl.BlockSpec((_HB, _W, _N), lambda i: (i, 0, 0)),
        ],
        out_specs=[
            pl.BlockSpec((1, 1), lambda i: (0, 0), memory_space=pltpu.SMEM),
            pl.BlockSpec((1, 1), lambda i: (0, 0), memory_space=pltpu.SMEM),
        ],
        out_shape=[
            jax.ShapeDtypeStruct((1, 1), jnp.float32),
            jax.ShapeDtypeStruct((1, 1), jnp.float32),
        ],
        scratch_shapes=[
            pltpu.VMEM((_G, _HB, _W, _N), jnp.int32),
            pltpu.VMEM((_HB, _W, _N), jnp.float32),
            pltpu.VMEM((_HB, _W, _N), jnp.int32),
            pltpu.VMEM((_HB, _W, _N), jnp.int32),
        ],
    )(scores, tgt)
    return loss[0, 0], acc[0, 0]
